# baseline (speedup 1.0000x reference)
import functools

import jax
import jax.numpy as jnp
from jax import lax
from jax.experimental import pallas as pl
from jax.experimental.pallas import tpu as pltpu
from jax.experimental.pallas import tpu_sc as plsc

NC = 2
NS = 16
LANES = 16
BATCH = 16384
B_TOTAL = 2 * BATCH
NW = NC * NS
B_PER_W = B_TOTAL // NW
CHUNK = 128
NCHUNK = B_PER_W // CHUNK
DIM = 64
COMB = 2 * DIM

_mesh = plsc.VectorSubcoreMesh(
    core_axis_name="c", subcore_axis_name="s", num_cores=NC, num_subcores=NS)

N_ROWS = 1000000
RB = 24576
HDIM = DIM // 2


def _words(blockT, eye, dn):
    lo = lax.dot_general(
        blockT[0:HDIM, :], eye, dn,
        preferred_element_type=jnp.float32).astype(jnp.bfloat16)
    hi = lax.dot_general(
        blockT[HDIM:DIM, :], eye, dn,
        preferred_element_type=jnp.float32).astype(jnp.bfloat16)
    lo32 = lax.bitcast_convert_type(lo, jnp.uint16).astype(jnp.uint32)
    hi32 = lax.bitcast_convert_type(hi, jnp.uint16).astype(jnp.uint32)
    return lax.bitcast_convert_type(
        lo32 | (hi32 << jnp.uint32(16)), jnp.int32)


RBH = 8192
NBLK = 62
HSPLIT = NBLK * RBH


def _repack_body(entA_ref, relA_ref, entB_ref, relB_ref, out_ref):
    eye = jnp.float32(1.0) * (
        lax.broadcasted_iota(jnp.int32, (HDIM, HDIM), 0)
        == lax.broadcasted_iota(jnp.int32, (HDIM, HDIM), 1))
    dn = (((0,), (0,)), ((), ()))
    out_ref[:, 0:32] = _words(entA_ref, eye, dn)
    out_ref[:, 32:64] = _words(relA_ref, eye, dn)
    out_ref[:, 64:96] = _words(entB_ref, eye, dn)
    out_ref[:, 96:128] = _words(relB_ref, eye, dn)


_repack_tc = pl.pallas_call(
    _repack_body,
    grid=(NBLK,),
    in_specs=[
        pl.BlockSpec((DIM, RBH), lambda g: (0, g)),
        pl.BlockSpec((DIM, RBH), lambda g: (0, g)),
        pl.BlockSpec((DIM, RBH), lambda g: (0, jnp.minimum(g + NBLK, 2 * NBLK - 2))),
        pl.BlockSpec((DIM, RBH), lambda g: (0, jnp.minimum(g + NBLK, 2 * NBLK - 2))),
    ],
    out_specs=pl.BlockSpec((RBH, COMB), lambda g: (g, 0)),
    out_shape=jax.ShapeDtypeStruct((HSPLIT, COMB), jnp.int32),
)


@functools.partial(
    pl.kernel,
    out_type=jax.ShapeDtypeStruct((B_TOTAL,), jnp.float32),
    mesh=_mesh,
    compiler_params=pltpu.CompilerParams(needs_layout_passes=False),
    scratch_types=[
        pltpu.VMEM((CHUNK,), jnp.int32),
        pltpu.VMEM((CHUNK,), jnp.int32),
        pltpu.VMEM((CHUNK,), jnp.int32),
        pltpu.VMEM((CHUNK,), jnp.int32),
        pltpu.VMEM((CHUNK,), jnp.int32),
        pltpu.VMEM((CHUNK,), jnp.int32),
        pltpu.VMEM((CHUNK, COMB), jnp.int32),
        pltpu.VMEM((CHUNK, COMB), jnp.int32),
        pltpu.VMEM((CHUNK, COMB), jnp.int32),
        pltpu.VMEM((LANES * CHUNK,), jnp.float32),
        pltpu.VMEM((CHUNK,), jnp.float32),
        pltpu.SemaphoreType.DMA,
    ],
)
def _transe_sc(comb_hbm, hidx_hbm, ridx_hbm, tidx_hbm, out_hbm,
               hpk_v, rpk_v, tpk_v, hsm, rsm, tsm,
               hrows_v, rrows_v, trows_v, part_v, outbuf_v, sem):
    wid = lax.axis_index("s") * NC + lax.axis_index("c")
    base = wid * B_PER_W
    lane_iota = lax.iota(jnp.int32, LANES)

    for k in range(NCHUNK):
        cb = base + k * CHUNK
        pltpu.sync_copy(hidx_hbm.at[pl.ds(cb, CHUNK)], hsm)
        pltpu.sync_copy(ridx_hbm.at[pl.ds(cb, CHUNK)], rsm)
        pltpu.sync_copy(tidx_hbm.at[pl.ds(cb, CHUNK)], tsm)
        for g in range(CHUNK // LANES):
            sl = pl.ds(g * LANES, LANES)
            hv, rv, tv = hsm[sl], rsm[sl], tsm[sl]
            hpk_v[sl] = jnp.where(hv >= HSPLIT, hv - HSPLIT, hv)
            rpk_v[sl] = jnp.where(rv >= HSPLIT, rv - HSPLIT, rv)
            tpk_v[sl] = jnp.where(tv >= HSPLIT, tv - HSPLIT, tv)
        ch = pltpu.async_copy(comb_hbm.at[hpk_v], hrows_v, sem)
        cr = pltpu.async_copy(comb_hbm.at[rpk_v], rrows_v, sem)
        ct = pltpu.async_copy(comb_hbm.at[tpk_v], trows_v, sem)
        ch.wait()
        cr.wait()
        ct.wait()

        def row_body(i, carry):
            bcast_i = jnp.full((LANES,), i, jnp.int32)
            hm = plsc.load_gather(hsm, [bcast_i]) >= HSPLIT
            rm = plsc.load_gather(rsm, [bcast_i]) >= HSPLIT
            tm = plsc.load_gather(tsm, [bcast_i]) >= HSPLIT
            acc = jnp.zeros((LANES,), jnp.float32)
            for c in range(HDIM // LANES):
                hw = jnp.where(hm, hrows_v[i, pl.ds(64 + c * LANES, LANES)],
                               hrows_v[i, pl.ds(c * LANES, LANES)])
                rw = jnp.where(rm, rrows_v[i, pl.ds(96 + c * LANES, LANES)],
                               rrows_v[i, pl.ds(32 + c * LANES, LANES)])
                tw = jnp.where(tm, trows_v[i, pl.ds(64 + c * LANES, LANES)],
                               trows_v[i, pl.ds(c * LANES, LANES)])
                hu = plsc.unpack(plsc.bitcast(hw, jnp.bfloat16),
                                 format=plsc.PackFormat.INTERLEAVED,
                                 preferred_element_type=jnp.float32)
                ru = plsc.unpack(plsc.bitcast(rw, jnp.bfloat16),
                                 format=plsc.PackFormat.INTERLEAVED,
                                 preferred_element_type=jnp.float32)
                tu = plsc.unpack(plsc.bitcast(tw, jnp.bfloat16),
                                 format=plsc.PackFormat.INTERLEAVED,
                                 preferred_element_type=jnp.float32)
                for p in range(2):
                    x = hu[p] + ru[p] - tu[p]
                    u = jnp.exp(x)
                    n = u * (u + 2.0)
                    y = x * (n / (n + 2.0))
                    acc = acc + y * y
            plsc.store_scatter(part_v, [lane_iota * CHUNK + i], acc)
            return carry

        lax.fori_loop(0, CHUNK, row_body, 0)

        for j in range(CHUNK // LANES):
            s = part_v[pl.ds(j * LANES, LANES)]
            for l in range(1, LANES):
                s = s + part_v[pl.ds(l * CHUNK + j * LANES, LANES)]
            seed = lax.shift_right_logical(
                plsc.bitcast(s, jnp.int32), 1) + jnp.int32(0x1FBD1DF5)
            t0 = plsc.bitcast(seed, jnp.float32)
            t0 = 0.5 * (t0 + s / t0)
            t0 = 0.5 * (t0 + s / t0)
            outbuf_v[pl.ds(j * LANES, LANES)] = t0

        pltpu.sync_copy(outbuf_v, out_hbm.at[pl.ds(cb, CHUNK)])


def kernel(positive_triplets, negative_triplets, offset, entities_emb, relations_emb):
    trip = jnp.concatenate([positive_triplets, negative_triplets], axis=0)
    hidx = trip[:, 0]
    ridx = trip[:, 1]
    tidx = trip[:, 2]
    entT = entities_emb.T
    relT = relations_emb.T
    comb = _repack_tc(entT, relT, entT, relT)
    dist = _transe_sc(comb, hidx, ridx, tidx)
    return (dist[:BATCH], dist[BATCH:])


# speedup vs baseline: 1.6708x; 1.6708x over previous
import functools

import jax
import jax.numpy as jnp
from jax import lax
from jax.experimental import pallas as pl
from jax.experimental.pallas import tpu as pltpu
from jax.experimental.pallas import tpu_sc as plsc

NC = 2
NS = 16
LANES = 16
BATCH = 16384
B_TOTAL = 2 * BATCH
NW = NC * NS
B_PER_W = B_TOTAL // NW
CHUNK = 128
NCHUNK = B_PER_W // CHUNK
DIM = 64
COMB = 2 * DIM

_mesh = plsc.VectorSubcoreMesh(
    core_axis_name="c", subcore_axis_name="s", num_cores=NC, num_subcores=NS)

N_ROWS = 1000000
RB = 20480


def _repack_body(entT_ref, relT_ref, out_ref):
    eye = jnp.float32(1.0) * (
        lax.broadcasted_iota(jnp.int32, (DIM, DIM), 0)
        == lax.broadcasted_iota(jnp.int32, (DIM, DIM), 1))
    dn = (((0,), (0,)), ((), ()))
    out_ref[:, 0:DIM] = lax.dot_general(
        entT_ref[...], eye, dn, preferred_element_type=jnp.float32)
    out_ref[:, DIM:COMB] = lax.dot_general(
        relT_ref[...], eye, dn, preferred_element_type=jnp.float32)


_repack_tc = pl.pallas_call(
    _repack_body,
    grid=(pl.cdiv(N_ROWS, RB),),
    in_specs=[
        pl.BlockSpec((DIM, RB), lambda g: (0, g)),
        pl.BlockSpec((DIM, RB), lambda g: (0, g)),
    ],
    out_specs=pl.BlockSpec((RB, COMB), lambda g: (g, 0)),
    out_shape=jax.ShapeDtypeStruct((N_ROWS, COMB), jnp.float32),
)


@functools.partial(
    pl.kernel,
    out_type=jax.ShapeDtypeStruct((B_TOTAL,), jnp.float32),
    mesh=_mesh,
    compiler_params=pltpu.CompilerParams(needs_layout_passes=False),
    scratch_types=[
        pltpu.VMEM((CHUNK,), jnp.int32),
        pltpu.VMEM((CHUNK,), jnp.int32),
        pltpu.VMEM((CHUNK,), jnp.int32),
        pltpu.VMEM((CHUNK, COMB), jnp.float32),
        pltpu.VMEM((CHUNK, COMB), jnp.float32),
        pltpu.VMEM((CHUNK, COMB), jnp.float32),
        pltpu.VMEM((LANES * CHUNK,), jnp.float32),
        pltpu.VMEM((CHUNK,), jnp.float32),
        pltpu.SemaphoreType.DMA,
    ],
)
def _transe_sc(comb_hbm, hidx_hbm, ridx_hbm, tidx_hbm, out_hbm,
               hidx_v, ridx_v, tidx_v, hrows_v, rrows_v, trows_v,
               part_v, outbuf_v, sem):
    wid = lax.axis_index("s") * NC + lax.axis_index("c")
    base = wid * B_PER_W
    lane_iota = lax.iota(jnp.int32, LANES)

    for k in range(NCHUNK):
        cb = base + k * CHUNK
        pltpu.sync_copy(hidx_hbm.at[pl.ds(cb, CHUNK)], hidx_v)
        pltpu.sync_copy(ridx_hbm.at[pl.ds(cb, CHUNK)], ridx_v)
        pltpu.sync_copy(tidx_hbm.at[pl.ds(cb, CHUNK)], tidx_v)
        ch = pltpu.async_copy(comb_hbm.at[hidx_v], hrows_v, sem)
        cr = pltpu.async_copy(comb_hbm.at[ridx_v], rrows_v, sem)
        ct = pltpu.async_copy(comb_hbm.at[tidx_v], trows_v, sem)
        ch.wait()
        cr.wait()
        ct.wait()

        def row_body(i, carry):
            acc = jnp.zeros((LANES,), jnp.float32)
            for c in range(DIM // LANES):
                h = hrows_v[i, pl.ds(c * LANES, LANES)]
                r = rrows_v[i, pl.ds(DIM + c * LANES, LANES)]
                t = trows_v[i, pl.ds(c * LANES, LANES)]
                x = h + r - t
                u = jnp.exp(x)
                n = u * (u + 2.0)
                y = x * (n / (n + 2.0))
                acc = acc + y * y
            plsc.store_scatter(part_v, [lane_iota * CHUNK + i], acc)
            return carry

        lax.fori_loop(0, CHUNK, row_body, 0)

        for j in range(CHUNK // LANES):
            s = part_v[pl.ds(j * LANES, LANES)]
            for l in range(1, LANES):
                s = s + part_v[pl.ds(l * CHUNK + j * LANES, LANES)]
            seed = lax.shift_right_logical(
                plsc.bitcast(s, jnp.int32), 1) + jnp.int32(0x1FBD1DF5)
            t0 = plsc.bitcast(seed, jnp.float32)
            t0 = 0.5 * (t0 + s / t0)
            t0 = 0.5 * (t0 + s / t0)
            outbuf_v[pl.ds(j * LANES, LANES)] = t0

        pltpu.sync_copy(outbuf_v, out_hbm.at[pl.ds(cb, CHUNK)])


def kernel(positive_triplets, negative_triplets, offset, entities_emb, relations_emb):
    trip = jnp.concatenate([positive_triplets, negative_triplets], axis=0)
    hidx = trip[:, 0]
    ridx = trip[:, 1]
    tidx = trip[:, 2]
    comb = _repack_tc(entities_emb.T, relations_emb.T)
    dist = _transe_sc(comb, hidx, ridx, tidx)
    return (dist[:BATCH], dist[BATCH:])


# speedup vs baseline: 1.6769x; 1.0037x over previous
import functools

import jax
import jax.numpy as jnp
from jax import lax
from jax.experimental import pallas as pl
from jax.experimental.pallas import tpu as pltpu
from jax.experimental.pallas import tpu_sc as plsc

NC = 2
NS = 16
LANES = 16
BATCH = 16384
B_TOTAL = 2 * BATCH
NW = NC * NS
B_PER_W = B_TOTAL // NW
CHUNK = 128
NCHUNK = B_PER_W // CHUNK
DIM = 64
COMB = 2 * DIM

_mesh = plsc.VectorSubcoreMesh(
    core_axis_name="c", subcore_axis_name="s", num_cores=NC, num_subcores=NS)

N_ROWS = 1000000
RB = 20480


def _repack_body(entT_ref, relT_ref, out_ref):
    out_ref[:, 0:DIM] = jnp.transpose(entT_ref[...])
    out_ref[:, DIM:COMB] = jnp.transpose(relT_ref[...])


_repack_tc = pl.pallas_call(
    _repack_body,
    grid=(pl.cdiv(N_ROWS, RB),),
    in_specs=[
        pl.BlockSpec((DIM, RB), lambda g: (0, g)),
        pl.BlockSpec((DIM, RB), lambda g: (0, g)),
    ],
    out_specs=pl.BlockSpec((RB, COMB), lambda g: (g, 0)),
    out_shape=jax.ShapeDtypeStruct((N_ROWS, COMB), jnp.float32),
)


@functools.partial(
    pl.kernel,
    out_type=jax.ShapeDtypeStruct((B_TOTAL,), jnp.float32),
    mesh=_mesh,
    compiler_params=pltpu.CompilerParams(needs_layout_passes=False),
    scratch_types=[
        pltpu.VMEM((CHUNK,), jnp.int32),
        pltpu.VMEM((CHUNK,), jnp.int32),
        pltpu.VMEM((CHUNK,), jnp.int32),
        pltpu.VMEM((CHUNK, COMB), jnp.float32),
        pltpu.VMEM((CHUNK, COMB), jnp.float32),
        pltpu.VMEM((CHUNK, COMB), jnp.float32),
        pltpu.VMEM((LANES * CHUNK,), jnp.float32),
        pltpu.VMEM((CHUNK,), jnp.float32),
        pltpu.SemaphoreType.DMA,
    ],
)
def _transe_sc(comb_hbm, hidx_hbm, ridx_hbm, tidx_hbm, out_hbm,
               hidx_v, ridx_v, tidx_v, hrows_v, rrows_v, trows_v,
               part_v, outbuf_v, sem):
    wid = lax.axis_index("s") * NC + lax.axis_index("c")
    base = wid * B_PER_W
    lane_iota = lax.iota(jnp.int32, LANES)

    for k in range(NCHUNK):
        cb = base + k * CHUNK
        pltpu.sync_copy(hidx_hbm.at[pl.ds(cb, CHUNK)], hidx_v)
        pltpu.sync_copy(ridx_hbm.at[pl.ds(cb, CHUNK)], ridx_v)
        pltpu.sync_copy(tidx_hbm.at[pl.ds(cb, CHUNK)], tidx_v)
        ch = pltpu.async_copy(comb_hbm.at[hidx_v], hrows_v, sem)
        cr = pltpu.async_copy(comb_hbm.at[ridx_v], rrows_v, sem)
        ct = pltpu.async_copy(comb_hbm.at[tidx_v], trows_v, sem)
        ch.wait()
        cr.wait()
        ct.wait()

        def row_body(i, carry):
            acc = jnp.zeros((LANES,), jnp.float32)
            for c in range(DIM // LANES):
                h = hrows_v[i, pl.ds(c * LANES, LANES)]
                r = rrows_v[i, pl.ds(DIM + c * LANES, LANES)]
                t = trows_v[i, pl.ds(c * LANES, LANES)]
                x = h + r - t
                u = jnp.exp(x)
                n = u * (u + 2.0)
                y = x * (n / (n + 2.0))
                acc = acc + y * y
            plsc.store_scatter(part_v, [lane_iota * CHUNK + i], acc)
            return carry

        lax.fori_loop(0, CHUNK, row_body, 0)

        for j in range(CHUNK // LANES):
            s = part_v[pl.ds(j * LANES, LANES)]
            for l in range(1, LANES):
                s = s + part_v[pl.ds(l * CHUNK + j * LANES, LANES)]
            seed = lax.shift_right_logical(
                plsc.bitcast(s, jnp.int32), 1) + jnp.int32(0x1FBD1DF5)
            t0 = plsc.bitcast(seed, jnp.float32)
            t0 = 0.5 * (t0 + s / t0)
            t0 = 0.5 * (t0 + s / t0)
            outbuf_v[pl.ds(j * LANES, LANES)] = t0

        pltpu.sync_copy(outbuf_v, out_hbm.at[pl.ds(cb, CHUNK)])


def kernel(positive_triplets, negative_triplets, offset, entities_emb, relations_emb):
    trip = jnp.concatenate([positive_triplets, negative_triplets], axis=0)
    hidx = trip[:, 0]
    ridx = trip[:, 1]
    tidx = trip[:, 2]
    comb = _repack_tc(entities_emb.T, relations_emb.T)
    dist = _transe_sc(comb, hidx, ridx, tidx)
    return (dist[:BATCH], dist[BATCH:])


# speedup vs baseline: 1.7489x; 1.0429x over previous
import functools

import jax
import jax.numpy as jnp
from jax import lax
from jax.experimental import pallas as pl
from jax.experimental.pallas import tpu as pltpu
from jax.experimental.pallas import tpu_sc as plsc

NC = 2
NS = 16
LANES = 16
BATCH = 16384
B_TOTAL = 2 * BATCH
NW = NC * NS
B_PER_W = B_TOTAL // NW
CHUNK = 128
NCHUNK = B_PER_W // CHUNK
DIM = 64
COMB = 2 * DIM

_mesh = plsc.VectorSubcoreMesh(
    core_axis_name="c", subcore_axis_name="s", num_cores=NC, num_subcores=NS)

N_ROWS = 1000000
RB = 20480


def _repack_body(entT_ref, relT_ref, out_ref):
    out_ref[:, 0:DIM] = jnp.transpose(entT_ref[...])
    out_ref[:, DIM:COMB] = jnp.transpose(relT_ref[...])


_repack_tc = pl.pallas_call(
    _repack_body,
    grid=(pl.cdiv(N_ROWS, RB),),
    in_specs=[
        pl.BlockSpec((DIM, RB), lambda g: (0, g)),
        pl.BlockSpec((DIM, RB), lambda g: (0, g)),
    ],
    out_specs=pl.BlockSpec((RB, COMB), lambda g: (g, 0)),
    out_shape=jax.ShapeDtypeStruct((N_ROWS, COMB), jnp.float32),
)


@functools.partial(
    pl.kernel,
    out_type=jax.ShapeDtypeStruct((B_TOTAL,), jnp.float32),
    mesh=_mesh,
    compiler_params=pltpu.CompilerParams(needs_layout_passes=False),
    scratch_types=[
        [[pltpu.VMEM((CHUNK,), jnp.int32)] * 3] * 2,
        [[pltpu.VMEM((CHUNK, COMB), jnp.float32)] * 3] * 2,
        pltpu.VMEM((LANES * CHUNK,), jnp.float32),
        pltpu.VMEM((CHUNK,), jnp.float32),
        [pltpu.SemaphoreType.DMA] * 2,
    ],
)
def _transe_sc(comb_hbm, hidx_hbm, ridx_hbm, tidx_hbm, out_hbm,
               idx_sets, row_sets, part_v, outbuf_v, sems):
    wid = lax.axis_index("s") * NC + lax.axis_index("c")
    base = wid * B_PER_W
    lane_iota = lax.iota(jnp.int32, LANES)

    def issue(k, slot):
        cb = base + k * CHUNK
        hidx_v, ridx_v, tidx_v = idx_sets[slot]
        hrows_v, rrows_v, trows_v = row_sets[slot]
        pltpu.sync_copy(hidx_hbm.at[pl.ds(cb, CHUNK)], hidx_v)
        pltpu.sync_copy(ridx_hbm.at[pl.ds(cb, CHUNK)], ridx_v)
        pltpu.sync_copy(tidx_hbm.at[pl.ds(cb, CHUNK)], tidx_v)
        pltpu.async_copy(comb_hbm.at[hidx_v], hrows_v, sems[slot])
        pltpu.async_copy(comb_hbm.at[ridx_v], rrows_v, sems[slot])
        pltpu.async_copy(comb_hbm.at[tidx_v], trows_v, sems[slot])

    issue(0, 0)
    for k in range(NCHUNK):
        slot = k % 2
        cb = base + k * CHUNK
        hrows_v, rrows_v, trows_v = row_sets[slot]
        if k + 1 < NCHUNK:
            issue(k + 1, 1 - slot)
        for b in (hrows_v, rrows_v, trows_v):
            pltpu.make_async_copy(
                comb_hbm.at[pl.ds(0, CHUNK)], b, sems[slot]).wait()

        def row_body(i, carry):
            acc = jnp.zeros((LANES,), jnp.float32)
            for c in range(DIM // LANES):
                h = hrows_v[i, pl.ds(c * LANES, LANES)]
                r = rrows_v[i, pl.ds(DIM + c * LANES, LANES)]
                t = trows_v[i, pl.ds(c * LANES, LANES)]
                x = h + r - t
                u = jnp.exp(x)
                n = u * (u + 2.0)
                y = x * (n / (n + 2.0))
                acc = acc + y * y
            plsc.store_scatter(part_v, [lane_iota * CHUNK + i], acc)
            return carry

        lax.fori_loop(0, CHUNK, row_body, 0)

        for j in range(CHUNK // LANES):
            s = part_v[pl.ds(j * LANES, LANES)]
            for l in range(1, LANES):
                s = s + part_v[pl.ds(l * CHUNK + j * LANES, LANES)]
            seed = lax.shift_right_logical(
                plsc.bitcast(s, jnp.int32), 1) + jnp.int32(0x1FBD1DF5)
            t0 = plsc.bitcast(seed, jnp.float32)
            t0 = 0.5 * (t0 + s / t0)
            t0 = 0.5 * (t0 + s / t0)
            outbuf_v[pl.ds(j * LANES, LANES)] = t0

        pltpu.sync_copy(outbuf_v, out_hbm.at[pl.ds(cb, CHUNK)])


def kernel(positive_triplets, negative_triplets, offset, entities_emb, relations_emb):
    trip = jnp.concatenate([positive_triplets, negative_triplets], axis=0)
    hidx = trip[:, 0]
    ridx = trip[:, 1]
    tidx = trip[:, 2]
    comb = _repack_tc(entities_emb.T, relations_emb.T)
    dist = _transe_sc(comb, hidx, ridx, tidx)
    return (dist[:BATCH], dist[BATCH:])
